# baseline (speedup 1.0000x reference)
import functools

import jax
import jax.numpy as jnp
from jax import lax
from jax.experimental import pallas as pl
from jax.experimental.pallas import tpu as pltpu
from jax.experimental.pallas import tpu_sc as plsc

N = 10000
D = 128
H = D // 2
E_L = 320000
E_G = 160000
NC, NS = 2, 16
EPT_L = E_L // NS
EPT_G = E_G // NS
K = 80
B = 5
STEPS_L = EPT_L // (K * B)
STEPS_G = EPT_G // (K * B)
RPT = 1248
RPT_LAST = 2 * N - (NS - 1) * RPT
BN_EPS = 1e-5


def _sc_aggregate(x2r, lei, gei, zrows):
    mesh = plsc.VectorSubcoreMesh(core_axis_name="c", subcore_axis_name="s",
                                  num_cores=NC, num_subcores=NS)

    @functools.partial(
        pl.kernel,
        out_type=jax.ShapeDtypeStruct((NC, 2 * N, H), jnp.float32),
        mesh=mesh,
        scratch_types=[
            pltpu.VMEM((B, 2, 2, K), jnp.int32),
            pltpu.VMEM((B, K, H), jnp.float32),
            pltpu.VMEM_SHARED((2 * N, H), jnp.float32),
            pltpu.SemaphoreType.DMA((B,)),
            pltpu.SemaphoreType.DMA((B,)),
            pltpu.SemaphoreType.DMA((B,)),
        ],
        compiler_params=pltpu.CompilerParams(use_tc_tiling_on_sc=False),
    )
    def k(x_hbm, lei_hbm, gei_hbm, z_hbm, out_hbm,
          ci, gb, acc, gsem, ssem, isem):
        cid = lax.axis_index("c")
        sid = lax.axis_index("s")

        def zero_acc():
            @pl.when(sid < NS - 1)
            def _():
                pltpu.sync_copy(z_hbm.at[pl.ds(0, RPT)],
                                acc.at[pl.ds(sid * RPT, RPT)])

            @pl.when(sid == NS - 1)
            def _():
                pltpu.sync_copy(z_hbm, acc.at[pl.ds((NS - 1) * RPT, RPT_LAST)])

            plsc.subcore_barrier()

        def load_idx(src, ebase, i, b, pp):
            start = pl.multiple_of(ebase + i * K, 8)
            pltpu.async_copy(src.at[1, pl.ds(start, K)], ci.at[b, pp, 0],
                             isem.at[b])
            pltpu.async_copy(src.at[0, pl.ds(start, K)], ci.at[b, pp, 1],
                             isem.at[b])

        def wait_idx_and_xform(src, ebase, i, b, pp, roff):
            start = pl.multiple_of(ebase + i * K, 8)
            pltpu.make_async_copy(src.at[1, pl.ds(start, K)], ci.at[b, pp, 0],
                                  isem.at[b]).wait()
            pltpu.make_async_copy(src.at[0, pl.ds(start, K)], ci.at[b, pp, 1],
                                  isem.at[b]).wait()
            for v in range(K // 16):
                sl = pl.ds(v * 16, 16)
                ci[b, pp, 0, sl] = ci[b, pp, 0, sl] * 2 + cid
                if roff:
                    ci[b, pp, 1, sl] = ci[b, pp, 1, sl] + roff

        def run_list(src, ebase, nsteps, roff, after_prime=None):
            for b in range(B):
                load_idx(src, ebase, b, b, 0)
            for b in range(B):
                wait_idx_and_xform(src, ebase, b, b, 0, roff)
                pltpu.async_copy(x_hbm.at[ci.at[b, 0, 0]], gb.at[b], gsem.at[b])
            if after_prime is not None:
                after_prime()

            def step(g, carry):
                p = lax.rem(g, 2)
                pn = 1 - p
                scat = []
                for b in range(B):
                    pltpu.make_async_copy(x_hbm.at[ci.at[b, p, 0]], gb.at[b],
                                          gsem.at[b]).wait()
                    scat.append(pltpu.async_copy(
                        gb.at[b], acc.at[ci.at[b, p, 1]], ssem.at[b], add=True))

                    @pl.when(g < nsteps - 1)
                    def _():
                        load_idx(src, ebase, (g + 1) * B + b, b, pn)

                for b in range(B):
                    scat[b].wait()

                    @pl.when(g < nsteps - 1)
                    def _():
                        wait_idx_and_xform(src, ebase, (g + 1) * B + b, b, pn,
                                           roff)
                        pltpu.async_copy(x_hbm.at[ci.at[b, pn, 0]], gb.at[b],
                                         gsem.at[b])

                return carry

            lax.fori_loop(0, nsteps, step, 0)

        run_list(lei_hbm, sid * EPT_L, STEPS_L, 0, after_prime=zero_acc)
        run_list(gei_hbm, sid * EPT_G, STEPS_G, N)

        plsc.subcore_barrier()

        @pl.when(sid < NS - 1)
        def _():
            pltpu.sync_copy(acc.at[pl.ds(sid * RPT, RPT)],
                            out_hbm.at[cid, pl.ds(sid * RPT, RPT)])

        @pl.when(sid == NS - 1)
        def _():
            pltpu.sync_copy(acc.at[pl.ds((NS - 1) * RPT, RPT_LAST)],
                            out_hbm.at[cid, pl.ds((NS - 1) * RPT, RPT_LAST)])

    return k(x2r, lei, gei, zrows)


def _tc_xw1_body(x_ref, w1t_ref, out_ref):
    out_ref[...] = jnp.dot(x_ref[...], w1t_ref[...],
                           preferred_element_type=jnp.float32)


def _tc_xw1(x, w1t):
    return pl.pallas_call(
        _tc_xw1_body,
        out_shape=jax.ShapeDtypeStruct((N, D), jnp.float32),
    )(x, w1t)


def _tc_finish_body(y1_ref, parts_ref, w2lt_ref, w2gt_ref,
                    gamma_ref, beta_ref, out_ref):
    f32 = jnp.float32
    out = y1_ref[...]
    out += jnp.dot(parts_ref[0, :N, :], w2lt_ref[:H, :], preferred_element_type=f32)
    out += jnp.dot(parts_ref[1, :N, :], w2lt_ref[H:, :], preferred_element_type=f32)
    out += jnp.dot(parts_ref[0, N:, :], w2gt_ref[:H, :], preferred_element_type=f32)
    out += jnp.dot(parts_ref[1, N:, :], w2gt_ref[H:, :], preferred_element_type=f32)
    mean = jnp.mean(out, axis=0, keepdims=True)
    var = jnp.mean(out * out, axis=0, keepdims=True) - mean * mean
    out = (out - mean) * lax.rsqrt(var + BN_EPS) * gamma_ref[...] + beta_ref[...]
    out_ref[...] = jnp.maximum(out, 0.0)


def _tc_finish(y1, parts, w2lt, w2gt, gamma2d, beta2d):
    return pl.pallas_call(
        _tc_finish_body,
        out_shape=jax.ShapeDtypeStruct((N, D), jnp.float32),
    )(y1, parts, w2lt, w2gt, gamma2d, beta2d)


def kernel(x, local_edge_index, global_edge_index, W1, W2_local, W2_global,
           gamma, beta):
    x2r = x.reshape(2 * N, H)
    zrows = lax.optimization_barrier(jnp.zeros((RPT_LAST, H), dtype=jnp.float32))

    parts = _sc_aggregate(x2r, local_edge_index, global_edge_index, zrows)
    y1 = _tc_xw1(x, W1.T)

    return _tc_finish(y1, parts, W2_local.T, W2_global.T,
                      gamma.reshape(1, D), beta.reshape(1, D))


# speedup vs baseline: 1.0053x; 1.0053x over previous
import functools

import jax
import jax.numpy as jnp
from jax import lax
from jax.experimental import pallas as pl
from jax.experimental.pallas import tpu as pltpu
from jax.experimental.pallas import tpu_sc as plsc

N = 10000
D = 128
H = D // 2
E_L = 320000
E_G = 160000
NC, NS = 2, 16
EPT_L = E_L // NS
EPT_G = E_G // NS
K = 80
B = 5
STEPS_L = EPT_L // (K * B)
STEPS_G = EPT_G // (K * B)
RPT = 1248
RPT_LAST = 2 * N - (NS - 1) * RPT
BN_EPS = 1e-5


def _sc_aggregate(x2r, lei, gei, zrows):
    mesh = plsc.VectorSubcoreMesh(core_axis_name="c", subcore_axis_name="s",
                                  num_cores=NC, num_subcores=NS)

    @functools.partial(
        pl.kernel,
        out_type=jax.ShapeDtypeStruct((NC, 2 * N, H), jnp.float32),
        mesh=mesh,
        scratch_types=[
            pltpu.VMEM((B, 2, 2, K), jnp.int32),
            pltpu.VMEM((B, K, H), jnp.float32),
            pltpu.VMEM_SHARED((2 * N, H), jnp.float32),
            pltpu.SemaphoreType.DMA((B,)),
            pltpu.SemaphoreType.DMA((B,)),
            pltpu.SemaphoreType.DMA((B,)),
        ],
        compiler_params=pltpu.CompilerParams(use_tc_tiling_on_sc=False),
    )
    def k(x_hbm, lei_hbm, gei_hbm, z_hbm, out_hbm,
          ci, gb, acc, gsem, ssem, isem):
        cid = lax.axis_index("c")
        sid = lax.axis_index("s")

        def zero_acc():
            @pl.when(sid < NS - 1)
            def _():
                pltpu.sync_copy(z_hbm.at[pl.ds(0, RPT)],
                                acc.at[pl.ds(sid * RPT, RPT)])

            @pl.when(sid == NS - 1)
            def _():
                pltpu.sync_copy(z_hbm, acc.at[pl.ds((NS - 1) * RPT, RPT_LAST)])

            plsc.subcore_barrier()

        def load_idx(src, ebase, i, b, pp):
            start = pl.multiple_of(ebase + i * K, 8)
            pltpu.async_copy(src.at[1, pl.ds(start, K)], ci.at[b, pp, 0],
                             isem.at[b])
            pltpu.async_copy(src.at[0, pl.ds(start, K)], ci.at[b, pp, 1],
                             isem.at[b])

        def wait_idx_and_xform(src, ebase, i, b, pp, roff):
            start = pl.multiple_of(ebase + i * K, 8)
            pltpu.make_async_copy(src.at[1, pl.ds(start, K)], ci.at[b, pp, 0],
                                  isem.at[b]).wait()
            pltpu.make_async_copy(src.at[0, pl.ds(start, K)], ci.at[b, pp, 1],
                                  isem.at[b]).wait()
            for v in range(K // 16):
                sl = pl.ds(v * 16, 16)
                ci[b, pp, 0, sl] = ci[b, pp, 0, sl] * 2 + cid
                if roff:
                    ci[b, pp, 1, sl] = ci[b, pp, 1, sl] + roff

        def run_list(src, ebase, nsteps, roff, after_prime=None):
            for b in range(B):
                load_idx(src, ebase, b, b, 0)
            for b in range(B):
                wait_idx_and_xform(src, ebase, b, b, 0, roff)
                pltpu.async_copy(x_hbm.at[ci.at[b, 0, 0]], gb.at[b], gsem.at[b])
            if after_prime is not None:
                after_prime()

            def step(g, carry):
                p = lax.rem(g, 2)
                pn = 1 - p
                scat = []
                for b in range(B):
                    pltpu.make_async_copy(x_hbm.at[ci.at[b, p, 0]], gb.at[b],
                                          gsem.at[b]).wait()
                    scat.append(pltpu.async_copy(
                        gb.at[b], acc.at[ci.at[b, p, 1]], ssem.at[b], add=True))

                    @pl.when(g < nsteps - 1)
                    def _():
                        load_idx(src, ebase, (g + 1) * B + b, b, pn)

                for b in range(B):
                    scat[b].wait()

                    @pl.when(g < nsteps - 1)
                    def _():
                        wait_idx_and_xform(src, ebase, (g + 1) * B + b, b, pn,
                                           roff)
                        pltpu.async_copy(x_hbm.at[ci.at[b, pn, 0]], gb.at[b],
                                         gsem.at[b])

                return carry

            lax.fori_loop(0, nsteps, step, 0)

        run_list(lei_hbm, sid * EPT_L, STEPS_L, 0, after_prime=zero_acc)
        run_list(gei_hbm, sid * EPT_G, STEPS_G, N)

        plsc.subcore_barrier()

        @pl.when(sid < NS - 1)
        def _():
            pltpu.sync_copy(acc.at[pl.ds(sid * RPT, RPT)],
                            out_hbm.at[cid, pl.ds(sid * RPT, RPT)])

        @pl.when(sid == NS - 1)
        def _():
            pltpu.sync_copy(acc.at[pl.ds((NS - 1) * RPT, RPT_LAST)],
                            out_hbm.at[cid, pl.ds((NS - 1) * RPT, RPT_LAST)])

    return k(x2r, lei, gei, zrows)


def _mm_t(a, w):
    return lax.dot_general(a, w, (((1,), (1,)), ((), ())),
                           preferred_element_type=jnp.float32)


def _tc_finish_body(x_ref, parts_ref, w1_ref, w2l_ref, w2g_ref,
                    gamma_ref, beta_ref, out_ref):
    out = _mm_t(x_ref[...], w1_ref[...])
    out += _mm_t(parts_ref[0, :N, :], w2l_ref[:, :H])
    out += _mm_t(parts_ref[1, :N, :], w2l_ref[:, H:])
    out += _mm_t(parts_ref[0, N:, :], w2g_ref[:, :H])
    out += _mm_t(parts_ref[1, N:, :], w2g_ref[:, H:])
    mean = jnp.mean(out, axis=0, keepdims=True)
    var = jnp.mean(out * out, axis=0, keepdims=True) - mean * mean
    out = (out - mean) * lax.rsqrt(var + BN_EPS) * gamma_ref[...] + beta_ref[...]
    out_ref[...] = jnp.maximum(out, 0.0)


def _tc_finish(x, parts, w1, w2l, w2g, gamma2d, beta2d):
    return pl.pallas_call(
        _tc_finish_body,
        out_shape=jax.ShapeDtypeStruct((N, D), jnp.float32),
    )(x, parts, w1, w2l, w2g, gamma2d, beta2d)


def kernel(x, local_edge_index, global_edge_index, W1, W2_local, W2_global,
           gamma, beta):
    x2r = x.reshape(2 * N, H)
    zrows = lax.optimization_barrier(jnp.zeros((RPT_LAST, H), dtype=jnp.float32))

    parts = _sc_aggregate(x2r, local_edge_index, global_edge_index, zrows)

    return _tc_finish(x, parts, W1, W2_local, W2_global,
                      gamma.reshape(1, D), beta.reshape(1, D))


# speedup vs baseline: 1.0446x; 1.0391x over previous
import functools

import jax
import jax.numpy as jnp
from jax import lax
from jax.experimental import pallas as pl
from jax.experimental.pallas import tpu as pltpu
from jax.experimental.pallas import tpu_sc as plsc

N = 10000
D = 128
H = D // 2
E_L = 320000
E_G = 160000
NC, NS = 2, 16
EPT_L = E_L // NS
EPT_G = E_G // NS
K = 80
B = 5
STEPS_L = EPT_L // (K * B)
STEPS_G = EPT_G // (K * B)
RPT = 1248
RPT_LAST = 2 * N - (NS - 1) * RPT
BN_EPS = 1e-5


def _sc_aggregate(x2r, lei, gei):
    mesh = plsc.VectorSubcoreMesh(core_axis_name="c", subcore_axis_name="s",
                                  num_cores=NC, num_subcores=NS)

    @functools.partial(
        pl.kernel,
        out_type=jax.ShapeDtypeStruct((NC, 2 * N, H), jnp.float32),
        mesh=mesh,
        scratch_types=[
            pltpu.VMEM((B, 2, 2, K), jnp.int32),
            pltpu.VMEM((B, K, H), jnp.float32),
            pltpu.VMEM((K, H), jnp.float32),
            pltpu.VMEM_SHARED((2 * N, H), jnp.float32),
            pltpu.SemaphoreType.DMA((B,)),
            pltpu.SemaphoreType.DMA((B,)),
            pltpu.SemaphoreType.DMA((B,)),
            pltpu.SemaphoreType.DMA,
        ],
        compiler_params=pltpu.CompilerParams(use_tc_tiling_on_sc=False),
    )
    def k(x_hbm, lei_hbm, gei_hbm, out_hbm,
          ci, gb, zb, acc, gsem, ssem, isem, zsem):
        cid = lax.axis_index("c")
        sid = lax.axis_index("s")

        def zero_acc():
            def zrow(r, carry):
                for v in range(H // 16):
                    zb[r, pl.ds(v * 16, 16)] = jnp.zeros((16,), jnp.float32)
                return carry

            lax.fori_loop(0, K, zrow, 0)
            base = sid * RPT
            for j in range(RPT // K):
                pltpu.async_copy(zb, acc.at[pl.ds(base + j * K, K)], zsem)
            for j in range(RPT // K):
                pltpu.make_async_copy(zb, acc.at[pl.ds(base + j * K, K)],
                                      zsem).wait()
            rem = RPT - (RPT // K) * K
            pltpu.sync_copy(zb.at[pl.ds(0, rem)],
                            acc.at[pl.ds(base + (RPT // K) * K, rem)])

            @pl.when(sid == NS - 1)
            def _():
                pltpu.sync_copy(zb.at[pl.ds(0, RPT_LAST - RPT)],
                                acc.at[pl.ds((NS - 1) * RPT + RPT, RPT_LAST - RPT)])

            plsc.subcore_barrier()

        def load_idx(src, ebase, i, b, pp):
            start = pl.multiple_of(ebase + i * K, 8)
            pltpu.async_copy(src.at[1, pl.ds(start, K)], ci.at[b, pp, 0],
                             isem.at[b])
            pltpu.async_copy(src.at[0, pl.ds(start, K)], ci.at[b, pp, 1],
                             isem.at[b])

        def wait_idx_and_xform(src, ebase, i, b, pp, roff):
            start = pl.multiple_of(ebase + i * K, 8)
            pltpu.make_async_copy(src.at[1, pl.ds(start, K)], ci.at[b, pp, 0],
                                  isem.at[b]).wait()
            pltpu.make_async_copy(src.at[0, pl.ds(start, K)], ci.at[b, pp, 1],
                                  isem.at[b]).wait()
            for v in range(K // 16):
                sl = pl.ds(v * 16, 16)
                ci[b, pp, 0, sl] = ci[b, pp, 0, sl] * 2 + cid
                if roff:
                    ci[b, pp, 1, sl] = ci[b, pp, 1, sl] + roff

        def run_list(src, ebase, nsteps, roff, after_prime=None):
            for b in range(B):
                load_idx(src, ebase, b, b, 0)
            for b in range(B):
                wait_idx_and_xform(src, ebase, b, b, 0, roff)
                pltpu.async_copy(x_hbm.at[ci.at[b, 0, 0]], gb.at[b], gsem.at[b])
            if after_prime is not None:
                after_prime()

            def step(g, carry):
                p = lax.rem(g, 2)
                pn = 1 - p
                scat = []
                for b in range(B):
                    @pl.when(g < nsteps - 1)
                    def _():
                        load_idx(src, ebase, (g + 1) * B + b, b, pn)

                    pltpu.make_async_copy(x_hbm.at[ci.at[b, p, 0]], gb.at[b],
                                          gsem.at[b]).wait()
                    scat.append(pltpu.async_copy(
                        gb.at[b], acc.at[ci.at[b, p, 1]], ssem.at[b], add=True))

                for b in range(B):
                    @pl.when(g < nsteps - 1)
                    def _():
                        wait_idx_and_xform(src, ebase, (g + 1) * B + b, b, pn,
                                           roff)

                    scat[b].wait()

                    @pl.when(g < nsteps - 1)
                    def _():
                        pltpu.async_copy(x_hbm.at[ci.at[b, pn, 0]], gb.at[b],
                                         gsem.at[b])

                return carry

            lax.fori_loop(0, nsteps, step, 0)

        run_list(lei_hbm, sid * EPT_L, STEPS_L, 0, after_prime=zero_acc)
        run_list(gei_hbm, sid * EPT_G, STEPS_G, N)

        plsc.subcore_barrier()

        @pl.when(sid < NS - 1)
        def _():
            pltpu.sync_copy(acc.at[pl.ds(sid * RPT, RPT)],
                            out_hbm.at[cid, pl.ds(sid * RPT, RPT)])

        @pl.when(sid == NS - 1)
        def _():
            pltpu.sync_copy(acc.at[pl.ds((NS - 1) * RPT, RPT_LAST)],
                            out_hbm.at[cid, pl.ds((NS - 1) * RPT, RPT_LAST)])

    return k(x2r, lei, gei)


def _mm_t(a, w):
    return lax.dot_general(a, w, (((1,), (1,)), ((), ())),
                           preferred_element_type=jnp.float32)


def _tc_finish_body(x_ref, parts_ref, w1_ref, w2l_ref, w2g_ref,
                    gamma_ref, beta_ref, out_ref):
    out = _mm_t(x_ref[...], w1_ref[...])
    out += _mm_t(parts_ref[0, :N, :], w2l_ref[:, :H])
    out += _mm_t(parts_ref[1, :N, :], w2l_ref[:, H:])
    out += _mm_t(parts_ref[0, N:, :], w2g_ref[:, :H])
    out += _mm_t(parts_ref[1, N:, :], w2g_ref[:, H:])
    mean = jnp.mean(out, axis=0, keepdims=True)
    var = jnp.mean(out * out, axis=0, keepdims=True) - mean * mean
    out = (out - mean) * lax.rsqrt(var + BN_EPS) * gamma_ref[...] + beta_ref[...]
    out_ref[...] = jnp.maximum(out, 0.0)


def _tc_finish(x, parts, w1, w2l, w2g, gamma2d, beta2d):
    return pl.pallas_call(
        _tc_finish_body,
        out_shape=jax.ShapeDtypeStruct((N, D), jnp.float32),
    )(x, parts, w1, w2l, w2g, gamma2d, beta2d)


def kernel(x, local_edge_index, global_edge_index, W1, W2_local, W2_global,
           gamma, beta):
    x2r = x.reshape(2 * N, H)
    parts = _sc_aggregate(x2r, local_edge_index, global_edge_index)

    return _tc_finish(x, parts, W1, W2_local, W2_global,
                      gamma.reshape(1, D), beta.reshape(1, D))
